# Initial kernel scaffold; baseline (speedup 1.0000x reference)
import jax
import jax.numpy as jnp
from jax.experimental import pallas as pl


def kernel(logits, labels):
    raise NotImplementedError("write your pallas kernel here")


# speedup vs baseline: 32.3463x; 32.3463x over previous
import functools

import jax
import jax.numpy as jnp
from jax.experimental import pallas as pl


_ROWS_PER_BLOCK = 256


def _score_block(logits_ref, labels_ref, out_ref):
    x = logits_ref[...]
    lab = labels_ref[...]
    m = jnp.max(x, axis=1, keepdims=True)
    e = jnp.exp(x - m)
    z = jnp.sum(e, axis=1, keepdims=True)
    p = e / z
    col = jax.lax.broadcasted_iota(jnp.int32, x.shape, 1)
    is_label = col == lab
    p_l = jnp.sum(jnp.where(is_label, p, 0.0), axis=1, keepdims=True)
    gt_sum = jnp.sum(jnp.where(p > p_l, p, 0.0), axis=1, keepdims=True)
    tie_cnt = jnp.sum(
        jnp.where((p == p_l) & (col <= lab), 1.0, 0.0), axis=1, keepdims=True
    )
    out_ref[...] = gt_sum + p_l * tie_cnt


@jax.jit
def kernel(logits, labels):
    n, c = logits.shape
    labels2d = labels.astype(jnp.int32).reshape(n, 1)
    br = _ROWS_PER_BLOCK
    grid = (n // br,)
    out = pl.pallas_call(
        _score_block,
        grid=grid,
        in_specs=[
            pl.BlockSpec((br, c), lambda i: (i, 0)),
            pl.BlockSpec((br, 1), lambda i: (i, 0)),
        ],
        out_specs=pl.BlockSpec((br, 1), lambda i: (i, 0)),
        out_shape=jax.ShapeDtypeStruct((n, 1), jnp.float32),
    )(logits, labels2d)
    return out[:, 0]
